# Initial kernel scaffold; baseline (speedup 1.0000x reference)
import jax
import jax.numpy as jnp
from jax.experimental import pallas as pl


def kernel(x, edge_index, Wl1, bl1, Wr1, Wl2, bl2, Wr2, Wl3, bl3, Wr3):
    raise NotImplementedError("write your pallas kernel here")


# speedup vs baseline: 1.0230x; 1.0230x over previous
import functools

import jax
import jax.numpy as jnp
from jax.experimental import pallas as pl

N = 10000
E = 320000
ROW_BLK = 2000


def _layer_body(aggr_ref, h_ref, wlT_ref, wrT_ref, bl_ref, o_ref, *, final):
    a = aggr_ref[...]
    a = jnp.where(jnp.isfinite(a), a, 0.0)
    acc = jnp.dot(a, wlT_ref[...], preferred_element_type=jnp.float32)
    acc += jnp.dot(h_ref[...], wrT_ref[...], preferred_element_type=jnp.float32)
    acc += bl_ref[...]
    if final:
        m = jnp.max(acc, axis=1, keepdims=True)
        z = acc - m
        lse = jnp.log(jnp.sum(jnp.exp(z), axis=1, keepdims=True))
        acc = z - lse
    o_ref[...] = acc


def _tc_layer(aggr, h, wlT, wrT, bl, *, final=False):
    fin = h.shape[1]
    hout = wlT.shape[1]
    grid = (N // ROW_BLK,)
    return pl.pallas_call(
        functools.partial(_layer_body, final=final),
        grid=grid,
        in_specs=[
            pl.BlockSpec((ROW_BLK, fin), lambda i: (i, 0)),
            pl.BlockSpec((ROW_BLK, fin), lambda i: (i, 0)),
            pl.BlockSpec((fin, hout), lambda i: (0, 0)),
            pl.BlockSpec((fin, hout), lambda i: (0, 0)),
            pl.BlockSpec((1, hout), lambda i: (0, 0)),
        ],
        out_specs=pl.BlockSpec((ROW_BLK, hout), lambda i: (i, 0)),
        out_shape=jax.ShapeDtypeStruct((N, hout), jnp.float32),
    )(aggr, h, wlT, wrT, bl)


def _pad2(a, r, c):
    return jnp.zeros((r, c), a.dtype).at[: a.shape[0], : a.shape[1]].set(a)


def _segmax(h, src, dst):
    msgs = jnp.take(h, src, axis=0)
    return jax.ops.segment_max(msgs, dst, num_segments=N)


def kernel(x, edge_index, Wl1, bl1, Wr1, Wl2, bl2, Wr2, Wl3, bl3, Wr3):
    src = edge_index[0]
    dst = edge_index[1]

    wlT1 = _pad2(Wl1.T, 128, 208)
    wrT1 = _pad2(Wr1.T, 128, 208)
    b1 = _pad2(bl1[None, :], 1, 208)
    wlT2 = _pad2(Wl2.T, 208, 64)
    wrT2 = _pad2(Wr2.T, 208, 64)
    b2 = _pad2(bl2[None, :], 1, 64)
    wlT3 = _pad2(Wl3.T, 64, 16)
    wrT3 = _pad2(Wr3.T, 64, 16)
    b3 = _pad2(bl3[None, :], 1, 16)

    aggr1 = _segmax(x, src, dst)
    h1 = _tc_layer(aggr1, x, wlT1, wrT1, b1)

    aggr2 = _segmax(h1, src, dst)
    h2 = _tc_layer(aggr2, h1, wlT2, wrT2, b2)

    aggr3 = _segmax(h2, src, dst)
    out = _tc_layer(aggr3, h2, wlT3, wrT3, b3, final=True)
    return out
